# Initial kernel scaffold; baseline (speedup 1.0000x reference)
import jax
import jax.numpy as jnp
from jax.experimental import pallas as pl


def kernel(x, Wg, bg):
    raise NotImplementedError("write your pallas kernel here")


# speedup vs baseline: 1.2192x; 1.2192x over previous
import functools

import jax
import jax.numpy as jnp
from jax.experimental import pallas as pl

T = 16384
D = 2048
TILE = 512
LANE_PAD = 128
THRESHOLD = 0.5


def _branch_route_body(x_ref, wg_ref, bg_ref, x0_ref, x1_ref, xout_ref):
    x = x_ref[...]
    z = jnp.dot(x, wg_ref[...], preferred_element_type=jnp.float32)
    s = jax.nn.sigmoid(z + bg_ref[...])
    a = jnp.where(s > THRESHOLD, s, 0.0)
    a0 = a[:, 0:1]
    a1 = a[:, 1:2]
    x0 = x * a0
    x1 = x * a1
    x0_ref[...] = x0
    x1_ref[...] = x1
    xout_ref[...] = x0 + x1


@jax.jit
def kernel(x, Wg, bg):
    wg_p = jnp.zeros((D, LANE_PAD), dtype=jnp.float32).at[:, :2].set(Wg)
    bg_p = jnp.zeros((1, LANE_PAD), dtype=jnp.float32).at[0, :2].set(bg)
    grid = (T // TILE,)
    out_shape = [jax.ShapeDtypeStruct((T, D), jnp.float32)] * 3
    x0, x1, xout = pl.pallas_call(
        _branch_route_body,
        grid=grid,
        in_specs=[
            pl.BlockSpec((TILE, D), lambda i: (i, 0)),
            pl.BlockSpec((D, LANE_PAD), lambda i: (0, 0)),
            pl.BlockSpec((1, LANE_PAD), lambda i: (0, 0)),
        ],
        out_specs=[pl.BlockSpec((TILE, D), lambda i: (i, 0))] * 3,
        out_shape=out_shape,
    )(x, wg_p, bg_p)
    return (x0, x1, xout)


# speedup vs baseline: 1.2468x; 1.0227x over previous
import functools

import jax
import jax.numpy as jnp
from jax.experimental import pallas as pl

T = 16384
D = 2048
TILE = 512
LANE_PAD = 128
THRESHOLD = 0.5


def _branch_route_body(x_ref, wg_ref, bg_ref, x0_ref, x1_ref, xout_ref):
    x = x_ref[...]
    wg = wg_ref[...]
    z0 = jnp.sum(x * wg[0:1, :], axis=1, keepdims=True)
    z1 = jnp.sum(x * wg[1:2, :], axis=1, keepdims=True)
    s0 = jax.nn.sigmoid(z0 + bg_ref[0, 0])
    s1 = jax.nn.sigmoid(z1 + bg_ref[0, 1])
    a0 = jnp.where(s0 > THRESHOLD, s0, 0.0)
    a1 = jnp.where(s1 > THRESHOLD, s1, 0.0)
    x0 = x * a0
    x1 = x * a1
    x0_ref[...] = x0
    x1_ref[...] = x1
    xout_ref[...] = x0 + x1


@jax.jit
def kernel(x, Wg, bg):
    wg_t = Wg.T
    bg_p = jnp.zeros((1, LANE_PAD), dtype=jnp.float32).at[0, :2].set(bg)
    grid = (T // TILE,)
    out_shape = [jax.ShapeDtypeStruct((T, D), jnp.float32)] * 3
    x0, x1, xout = pl.pallas_call(
        _branch_route_body,
        grid=grid,
        in_specs=[
            pl.BlockSpec((TILE, D), lambda i: (i, 0)),
            pl.BlockSpec((2, D), lambda i: (0, 0)),
            pl.BlockSpec((1, LANE_PAD), lambda i: (0, 0)),
        ],
        out_specs=[pl.BlockSpec((TILE, D), lambda i: (i, 0))] * 3,
        out_shape=out_shape,
    )(x, wg_t, bg_p)
    return (x0, x1, xout)
